# baseline (device time: 258655 ns/iter reference)
import jax
import jax.numpy as jnp
from jax import lax
from jax.experimental import pallas as pl
from jax.experimental.pallas import tpu as pltpu

_CH = 512
_NC = 32
_NDEV = 8
_PART = 1024
_HALF = 512
_NRING_CH = 16
_NX = 18

def kernel(x):
    m, n = x.shape
    n_half = n // 2

    def body(x_ref, out_ref, vf32, vsend, vkeep, load_sems, store_sems,
             xsend_sems, xrecv_sems, cw_send_sems, cw_recv_sems,
             ccw_send_sems, ccw_recv_sems):
        my_x = lax.axis_index("x")
        my_y = lax.axis_index("y")
        my_z = lax.axis_index("z")
        other = 1 - my_x

        my_r = jnp.where(my_y == 0, my_z, 7 - my_z)
        ny = jnp.where(my_y == 0, jnp.where(my_z < 3, 0, 1),
                       jnp.where(my_z > 0, 1, 0))
        nz = jnp.where(my_y == 0, jnp.where(my_z < 3, my_z + 1, 3),
                       jnp.where(my_z > 0, my_z - 1, 0))
        py = jnp.where(my_y == 0, jnp.where(my_z > 0, 0, 1),
                       jnp.where(my_z < 3, 1, 0))
        pz = jnp.where(my_y == 0, jnp.where(my_z > 0, my_z - 1, 0),
                       jnp.where(my_z < 3, my_z + 1, 3))
        nxt = (my_x, ny, nz)
        prv = (my_x, py, pz)
        par = (other, my_y, my_z)

        barrier_sem = pltpu.get_barrier_semaphore()
        for nbr in (par, nxt, prv):
            pl.semaphore_signal(
                barrier_sem, inc=1, device_id=nbr,
                device_id_type=pl.DeviceIdType.MESH,
            )
        pl.semaphore_wait(barrier_sem, 3)

        def rows0(i):
            if i < 2:
                return (2 * my_r + i) * _CH
            if i < _NX:
                return (14 + i) * _CH
            return ((2 * my_r + (i - 16)) % _NRING_CH) * _CH

        def load(i):
            return pltpu.make_async_copy(
                x_ref.at[pl.ds(rows0(i), _CH), :], vf32.at[i % 2],
                load_sems.at[i % 2],
            )

        stores = []
        xrdmas = []

        def process(i):
            load(i).wait()
            if i + 1 < _NC:
                load(i + 1).start()
            if i < _NX:
                if i >= 4:
                    xrdmas[i - 4].wait_send()
                vsend[i % 4] = vf32[i % 2, :, pl.ds(other * n_half, n_half)
                                    ].astype(jnp.bfloat16)
                xr = pltpu.make_async_remote_copy(
                    src_ref=vsend.at[i % 4],
                    dst_ref=out_ref.at[pl.ds(my_x * m + rows0(i), _CH), :],
                    send_sem=xsend_sems.at[i],
                    recv_sem=xrecv_sems.at[i],
                    device_id=par,
                    device_id_type=pl.DeviceIdType.MESH,
                )
                xr.start()
                xrdmas.append(xr)
            if i >= 2:
                stores[i - 2].wait()
            vkeep[i % 2] = vf32[i % 2, :, pl.ds(my_x * n_half, n_half)
                                ].astype(jnp.bfloat16)
            st = pltpu.make_async_copy(
                vkeep.at[i % 2],
                out_ref.at[pl.ds(my_x * m + rows0(i), _CH), :],
                store_sems.at[i],
            )
            st.start()
            stores.append(st)

        load(0).start()
        for i in range(2):
            process(i)

        def ring_rdma(j_part, half, dev, send_sem, recv_sem):
            rows = other * m + j_part * _PART + half * _HALF
            return pltpu.make_async_remote_copy(
                src_ref=out_ref.at[pl.ds(rows, _HALF), :],
                dst_ref=out_ref.at[pl.ds(rows, _HALF), :],
                send_sem=send_sem,
                recv_sem=recv_sem,
                device_id=dev,
                device_id_type=pl.DeviceIdType.MESH,
            )

        cw_recvs, ccw_recvs = [], []
        for s in range(_NDEV - 1):
            cw_recvs.append(ring_rdma((my_r - 1 - s) % 8, 0, prv,
                                      cw_send_sems.at[s], cw_recv_sems.at[s]))
            ccw_recvs.append(ring_rdma((my_r + 1 + s) % 8, 1, nxt,
                                       ccw_send_sems.at[s],
                                       ccw_recv_sems.at[s]))

        step_ranges = [(2, 7), (7, 12), (12, 17), (17, 21), (21, 25),
                       (25, 29), (29, 32)]
        cw_sends, ccw_sends = [], []
        for s in range(_NDEV - 1):
            if s == 0:
                xrdmas[0].wait_recv()
            else:
                cw_recvs[s - 1].wait_recv()
            cw = ring_rdma((my_r - s) % 8, 0, nxt,
                           cw_send_sems.at[s], cw_recv_sems.at[s])
            cw.start()
            cw_sends.append(cw)

            if s == 0:
                xrdmas[1].wait_recv()
            else:
                ccw_recvs[s - 1].wait_recv()
            ccw = ring_rdma((my_r + s) % 8, 1, prv,
                            ccw_send_sems.at[s], ccw_recv_sems.at[s])
            ccw.start()
            ccw_sends.append(ccw)

            for i in range(*step_ranges[s]):
                process(i)

        for i in range(2, _NX):
            xrdmas[i].wait_recv()
        cw_recvs[_NDEV - 2].wait_recv()
        ccw_recvs[_NDEV - 2].wait_recv()
        stores[_NC - 2].wait()
        stores[_NC - 1].wait()
        for i in range(_NX - 4, _NX):
            xrdmas[i].wait_send()
        for r in cw_sends + ccw_sends:
            r.wait_send()

    return pl.pallas_call(
        body,
        out_shape=jax.ShapeDtypeStruct((2 * m, n_half), jnp.bfloat16),
        in_specs=[pl.BlockSpec(memory_space=pltpu.MemorySpace.HBM)],
        out_specs=pl.BlockSpec(memory_space=pltpu.MemorySpace.HBM),
        scratch_shapes=[
            pltpu.VMEM((2, _CH, n), jnp.float32),
            pltpu.VMEM((4, _CH, n_half), jnp.bfloat16),
            pltpu.VMEM((2, _CH, n_half), jnp.bfloat16),
            pltpu.SemaphoreType.DMA((2,)),
            pltpu.SemaphoreType.DMA((_NC,)),
            pltpu.SemaphoreType.DMA((_NX,)),
            pltpu.SemaphoreType.DMA((_NX,)),
            pltpu.SemaphoreType.DMA((_NDEV - 1,)),
            pltpu.SemaphoreType.DMA((_NDEV - 1,)),
            pltpu.SemaphoreType.DMA((_NDEV - 1,)),
            pltpu.SemaphoreType.DMA((_NDEV - 1,)),
        ],
        compiler_params=pltpu.CompilerParams(collective_id=0),
    )(x)


# device time: 229648 ns/iter; 1.1263x vs baseline; 1.1263x over previous
import jax
import jax.numpy as jnp
from jax import lax
from jax.experimental import pallas as pl
from jax.experimental.pallas import tpu as pltpu

_CH = 512
_NC = 32
_NDEV = 8
_PART = 1536
_HALF = 768
_NRING_CH = 24
_NX = 11

def kernel(x):
    m, n = x.shape
    n_half = n // 2

    def body(x_ref, out_ref, vf32, vsend, vkeep, load_sems, store_sems,
             xsend_sems, xrecv_sems, cw_send_sems, cw_recv_sems,
             ccw_send_sems, ccw_recv_sems):
        my_x = lax.axis_index("x")
        my_y = lax.axis_index("y")
        my_z = lax.axis_index("z")
        other = 1 - my_x

        my_r = jnp.where(my_y == 0, my_z, 7 - my_z)
        ny = jnp.where(my_y == 0, jnp.where(my_z < 3, 0, 1),
                       jnp.where(my_z > 0, 1, 0))
        nz = jnp.where(my_y == 0, jnp.where(my_z < 3, my_z + 1, 3),
                       jnp.where(my_z > 0, my_z - 1, 0))
        py = jnp.where(my_y == 0, jnp.where(my_z > 0, 0, 1),
                       jnp.where(my_z < 3, 1, 0))
        pz = jnp.where(my_y == 0, jnp.where(my_z > 0, my_z - 1, 0),
                       jnp.where(my_z < 3, my_z + 1, 3))
        nxt = (my_x, ny, nz)
        prv = (my_x, py, pz)
        par = (other, my_y, my_z)

        barrier_sem = pltpu.get_barrier_semaphore()
        for nbr in (par, nxt, prv):
            pl.semaphore_signal(
                barrier_sem, inc=1, device_id=nbr,
                device_id_type=pl.DeviceIdType.MESH,
            )
        pl.semaphore_wait(barrier_sem, 3)

        def rows0(i):
            if i < 3:
                return (3 * my_r + i) * _CH
            if i < 11:
                return (21 + i) * _CH
            return ((3 * my_r + (i - 8)) % _NRING_CH) * _CH

        def load(i):
            return pltpu.make_async_copy(
                x_ref.at[pl.ds(rows0(i), _CH), :], vf32.at[i % 2],
                load_sems.at[i % 2],
            )

        stores = []
        xrdmas = []

        def process(i):
            load(i).wait()
            if i + 1 < _NC:
                load(i + 1).start()
            if i < _NX:
                if i >= 4:
                    xrdmas[i - 4].wait_send()
                vsend[i % 4] = vf32[i % 2, :, pl.ds(other * n_half, n_half)
                                    ].astype(jnp.bfloat16)
                xr = pltpu.make_async_remote_copy(
                    src_ref=vsend.at[i % 4],
                    dst_ref=out_ref.at[pl.ds(my_x * m + rows0(i), _CH), :],
                    send_sem=xsend_sems.at[i],
                    recv_sem=xrecv_sems.at[i],
                    device_id=par,
                    device_id_type=pl.DeviceIdType.MESH,
                )
                xr.start()
                xrdmas.append(xr)
            if i >= 2:
                stores[i - 2].wait()
            vkeep[i % 2] = vf32[i % 2, :, pl.ds(my_x * n_half, n_half)
                                ].astype(jnp.bfloat16)
            st = pltpu.make_async_copy(
                vkeep.at[i % 2],
                out_ref.at[pl.ds(my_x * m + rows0(i), _CH), :],
                store_sems.at[i],
            )
            st.start()
            stores.append(st)

        load(0).start()
        for i in range(3):
            process(i)

        def ring_rdma(j_part, half, dev, send_sem, recv_sem):
            rows = other * m + j_part * _PART + half * _HALF
            return pltpu.make_async_remote_copy(
                src_ref=out_ref.at[pl.ds(rows, _HALF), :],
                dst_ref=out_ref.at[pl.ds(rows, _HALF), :],
                send_sem=send_sem,
                recv_sem=recv_sem,
                device_id=dev,
                device_id_type=pl.DeviceIdType.MESH,
            )

        cw_recvs, ccw_recvs = [], []
        for s in range(_NDEV - 1):
            cw_recvs.append(ring_rdma((my_r - 1 - s) % 8, 0, prv,
                                      cw_send_sems.at[s], cw_recv_sems.at[s]))
            ccw_recvs.append(ring_rdma((my_r + 1 + s) % 8, 1, nxt,
                                       ccw_send_sems.at[s],
                                       ccw_recv_sems.at[s]))

        cw_sends, ccw_sends = [], []
        for s in range(_NDEV - 1):
            if s == 0:
                xrdmas[0].wait_recv()
                xrdmas[1].wait_recv()
            else:
                cw_recvs[s - 1].wait_recv()
            cw = ring_rdma((my_r - s) % 8, 0, nxt,
                           cw_send_sems.at[s], cw_recv_sems.at[s])
            cw.start()
            cw_sends.append(cw)

            if s == 0:
                xrdmas[2].wait_recv()
            else:
                ccw_recvs[s - 1].wait_recv()
            ccw = ring_rdma((my_r + s) % 8, 1, prv,
                            ccw_send_sems.at[s], ccw_recv_sems.at[s])
            ccw.start()
            ccw_sends.append(ccw)

            lo = 3 + 4 * s
            hi = min(lo + 4, _NC) if s < _NDEV - 2 else _NC
            for i in range(lo, hi):
                process(i)

        for i in range(3, _NX):
            xrdmas[i].wait_recv()
        cw_recvs[_NDEV - 2].wait_recv()
        ccw_recvs[_NDEV - 2].wait_recv()
        stores[_NC - 2].wait()
        stores[_NC - 1].wait()
        for i in range(_NX - 4, _NX):
            xrdmas[i].wait_send()
        for r in cw_sends + ccw_sends:
            r.wait_send()

    return pl.pallas_call(
        body,
        out_shape=jax.ShapeDtypeStruct((2 * m, n_half), jnp.bfloat16),
        in_specs=[pl.BlockSpec(memory_space=pltpu.MemorySpace.HBM)],
        out_specs=pl.BlockSpec(memory_space=pltpu.MemorySpace.HBM),
        scratch_shapes=[
            pltpu.VMEM((2, _CH, n), jnp.float32),
            pltpu.VMEM((4, _CH, n_half), jnp.bfloat16),
            pltpu.VMEM((2, _CH, n_half), jnp.bfloat16),
            pltpu.SemaphoreType.DMA((2,)),
            pltpu.SemaphoreType.DMA((_NC,)),
            pltpu.SemaphoreType.DMA((_NX,)),
            pltpu.SemaphoreType.DMA((_NX,)),
            pltpu.SemaphoreType.DMA((_NDEV - 1,)),
            pltpu.SemaphoreType.DMA((_NDEV - 1,)),
            pltpu.SemaphoreType.DMA((_NDEV - 1,)),
            pltpu.SemaphoreType.DMA((_NDEV - 1,)),
        ],
        compiler_params=pltpu.CompilerParams(collective_id=0),
    )(x)


# device time: 218147 ns/iter; 1.1857x vs baseline; 1.0527x over previous
import jax
import jax.numpy as jnp
from jax import lax
from jax.experimental import pallas as pl
from jax.experimental.pallas import tpu as pltpu

_CH = 512
_NC = 32
_NDEV = 8
_PART = 1536
_HALF = 768
_NRING_CH = 24
_NX = 11
_PIECE = 384

def kernel(x):
    m, n = x.shape
    n_half = n // 2

    def body(x_ref, out_ref, vf32, vsend, vkeep, load_sems, store_sems,
             xsend_sems, xrecv_sems, cw_send_sems, cw_recv_sems,
             ccw_send_sems, ccw_recv_sems):
        my_x = lax.axis_index("x")
        my_y = lax.axis_index("y")
        my_z = lax.axis_index("z")
        other = 1 - my_x

        my_r = jnp.where(my_y == 0, my_z, 7 - my_z)
        ny = jnp.where(my_y == 0, jnp.where(my_z < 3, 0, 1),
                       jnp.where(my_z > 0, 1, 0))
        nz = jnp.where(my_y == 0, jnp.where(my_z < 3, my_z + 1, 3),
                       jnp.where(my_z > 0, my_z - 1, 0))
        py = jnp.where(my_y == 0, jnp.where(my_z > 0, 0, 1),
                       jnp.where(my_z < 3, 1, 0))
        pz = jnp.where(my_y == 0, jnp.where(my_z > 0, my_z - 1, 0),
                       jnp.where(my_z < 3, my_z + 1, 3))
        nxt = (my_x, ny, nz)
        prv = (my_x, py, pz)
        par = (other, my_y, my_z)

        barrier_sem = pltpu.get_barrier_semaphore()
        for nbr in (par, nxt, prv):
            pl.semaphore_signal(
                barrier_sem, inc=1, device_id=nbr,
                device_id_type=pl.DeviceIdType.MESH,
            )
        pl.semaphore_wait(barrier_sem, 3)

        def rows0(i):
            if i < 3:
                return (3 * my_r + i) * _CH
            if i < 11:
                return (21 + i) * _CH
            return ((3 * my_r + (i - 8)) % _NRING_CH) * _CH

        def load(i):
            return pltpu.make_async_copy(
                x_ref.at[pl.ds(rows0(i), _CH), :], vf32.at[i % 2],
                load_sems.at[i % 2],
            )

        stores = []
        xrdmas = []

        def process(i):
            load(i).wait()
            if i + 1 < _NC:
                load(i + 1).start()
            if i < _NX:
                if i >= 4:
                    xrdmas[i - 4].wait_send()
                vsend[i % 4] = vf32[i % 2, :, pl.ds(other * n_half, n_half)
                                    ].astype(jnp.bfloat16)
                xr = pltpu.make_async_remote_copy(
                    src_ref=vsend.at[i % 4],
                    dst_ref=out_ref.at[pl.ds(my_x * m + rows0(i), _CH), :],
                    send_sem=xsend_sems.at[i],
                    recv_sem=xrecv_sems.at[i],
                    device_id=par,
                    device_id_type=pl.DeviceIdType.MESH,
                )
                xr.start()
                xrdmas.append(xr)
            if i >= 2:
                stores[i - 2].wait()
            vkeep[i % 2] = vf32[i % 2, :, pl.ds(my_x * n_half, n_half)
                                ].astype(jnp.bfloat16)
            st = pltpu.make_async_copy(
                vkeep.at[i % 2],
                out_ref.at[pl.ds(my_x * m + rows0(i), _CH), :],
                store_sems.at[i],
            )
            st.start()
            stores.append(st)

        load(0).start()
        for i in range(3):
            process(i)

        def ring_rdma(j_part, row_off, dev, send_sem, recv_sem):
            rows = other * m + j_part * _PART + row_off
            return pltpu.make_async_remote_copy(
                src_ref=out_ref.at[pl.ds(rows, _PIECE), :],
                dst_ref=out_ref.at[pl.ds(rows, _PIECE), :],
                send_sem=send_sem,
                recv_sem=recv_sem,
                device_id=dev,
                device_id_type=pl.DeviceIdType.MESH,
            )

        cw_recvs, ccw_recvs = [], []
        for s in range(_NDEV - 1):
            for p in range(2):
                k = 2 * s + p
                cw_recvs.append(
                    ring_rdma((my_r - 1 - s) % 8, p * _PIECE, prv,
                              cw_send_sems.at[k], cw_recv_sems.at[k]))
                ccw_recvs.append(
                    ring_rdma((my_r + 1 + s) % 8, _HALF + p * _PIECE, nxt,
                              ccw_send_sems.at[k], ccw_recv_sems.at[k]))

        cw_sends, ccw_sends = [], []
        for s in range(_NDEV - 1):
            if s == 0:
                xrdmas[0].wait_recv()
                cw0 = ring_rdma(my_r, 0, nxt,
                                cw_send_sems.at[0], cw_recv_sems.at[0])
                cw0.start()
                xrdmas[1].wait_recv()
                cw1 = ring_rdma(my_r, _PIECE, nxt,
                                cw_send_sems.at[1], cw_recv_sems.at[1])
                cw1.start()
                cw_sends += [cw0, cw1]
                xrdmas[2].wait_recv()
                for p in range(2):
                    ccw = ring_rdma(my_r, _HALF + p * _PIECE, prv,
                                    ccw_send_sems.at[p], ccw_recv_sems.at[p])
                    ccw.start()
                    ccw_sends.append(ccw)
            else:
                for p in range(2):
                    k = 2 * s + p
                    cw_recvs[k - 2].wait_recv()
                    cw = ring_rdma((my_r - s) % 8, p * _PIECE, nxt,
                                   cw_send_sems.at[k], cw_recv_sems.at[k])
                    cw.start()
                    cw_sends.append(cw)
                    ccw_recvs[k - 2].wait_recv()
                    ccw = ring_rdma((my_r + s) % 8, _HALF + p * _PIECE, prv,
                                    ccw_send_sems.at[k], ccw_recv_sems.at[k])
                    ccw.start()
                    ccw_sends.append(ccw)

            lo = 3 + 4 * s
            hi = min(lo + 4, _NC) if s < _NDEV - 2 else _NC
            for i in range(lo, hi):
                process(i)

        for i in range(3, _NX):
            xrdmas[i].wait_recv()
        for k in (2 * _NDEV - 4, 2 * _NDEV - 3):
            cw_recvs[k].wait_recv()
            ccw_recvs[k].wait_recv()
        stores[_NC - 2].wait()
        stores[_NC - 1].wait()
        for i in range(_NX - 4, _NX):
            xrdmas[i].wait_send()
        for r in cw_sends + ccw_sends:
            r.wait_send()

    return pl.pallas_call(
        body,
        out_shape=jax.ShapeDtypeStruct((2 * m, n_half), jnp.bfloat16),
        in_specs=[pl.BlockSpec(memory_space=pltpu.MemorySpace.HBM)],
        out_specs=pl.BlockSpec(memory_space=pltpu.MemorySpace.HBM),
        scratch_shapes=[
            pltpu.VMEM((2, _CH, n), jnp.float32),
            pltpu.VMEM((4, _CH, n_half), jnp.bfloat16),
            pltpu.VMEM((2, _CH, n_half), jnp.bfloat16),
            pltpu.SemaphoreType.DMA((2,)),
            pltpu.SemaphoreType.DMA((_NC,)),
            pltpu.SemaphoreType.DMA((_NX,)),
            pltpu.SemaphoreType.DMA((_NX,)),
            pltpu.SemaphoreType.DMA((2 * (_NDEV - 1),)),
            pltpu.SemaphoreType.DMA((2 * (_NDEV - 1),)),
            pltpu.SemaphoreType.DMA((2 * (_NDEV - 1),)),
            pltpu.SemaphoreType.DMA((2 * (_NDEV - 1),)),
        ],
        compiler_params=pltpu.CompilerParams(collective_id=0),
    )(x)
